# baseline (speedup 1.0000x reference)
import functools

import jax
import jax.numpy as jnp
from jax import lax
from jax.experimental import pallas as pl
from jax.experimental.pallas import tpu as pltpu
from jax.experimental.pallas import tpu_sc as plsc

_LATENTS = (32, 64, 96, 128, 192, 256)
_NE = 6
_BATCH = 16384
_IN_RAW = 73
_IN = 128
_DMAX = 256
_BB = 2048
_NBLK = _BATCH // _BB + (_NE - 1)
_P = _NBLK * _BB

_NC, _NS = 2, 16
_NW = _NC * _NS
_RPW = _BATCH // _NW
_RCHUNK = 128
_NCHUNK = _RPW // _RCHUNK


def _routing_body(roh_ref, pos_ref, be_ref):
    r = [roh_ref[:, e, :] for e in range(_NE)]
    m = r[0]
    for e in range(1, _NE):
        m = jnp.maximum(m, r[e])
    rate = jnp.full((128, 128), _NE, jnp.int32)
    for e in range(_NE - 1, -1, -1):
        rate = jnp.where(r[e] == m, e, rate)

    row_i = lax.broadcasted_iota(jnp.int32, (128, 128), 0)
    col_i = lax.broadcasted_iota(jnp.int32, (128, 128), 1)
    tri_incl = (row_i <= col_i).astype(jnp.float32)
    tri_strict = (col_i < row_i).astype(jnp.float32)

    ohs, ranks, cnts = [], [], []
    for e in range(_NE):
        oh = (rate == e).astype(jnp.float32)
        incl = jnp.dot(oh, tri_incl, preferred_element_type=jnp.float32)
        rowtot = jnp.sum(oh, axis=1, keepdims=True)
        prev_rows = jnp.dot(tri_strict, rowtot, preferred_element_type=jnp.float32)
        ranks.append(incl - oh + prev_rows)
        ohs.append(oh)
        cnts.append(jnp.sum(oh))

    starts = []
    sb = 0
    for e in range(_NE):
        starts.append(sb)
        nb = (cnts[e].astype(jnp.int32) + _BB - 1) // _BB
        sb = sb + nb

    pos = jnp.zeros((128, 128), jnp.float32)
    for e in range(_NE):
        off = (starts[e] * _BB).astype(jnp.float32) if e else jnp.float32(0)
        pos = pos + ohs[e] * (ranks[e] + off)
    pos_ref[...] = pos.astype(jnp.int32)

    bi = lax.broadcasted_iota(jnp.int32, (8, 128), 1)
    be = jnp.full((8, 128), -1, jnp.int32)
    for e in range(_NE):
        se = starts[e] if e else jnp.int32(0)
        be = be + jnp.where(bi >= se, 1, 0)
    be_ref[...] = be


def _route(roh3):
    return pl.pallas_call(
        _routing_body,
        out_shape=(
            jax.ShapeDtypeStruct((128, 128), jnp.int32),
            jax.ShapeDtypeStruct((8, 128), jnp.int32),
        ),
    )(roh3)


def _mlp_body(gid_ref, d_ref, a1_ref, a2_ref, x_ref,
              w1_ref, b1_ref, w2_ref, b2_ref, w3_ref, b3_ref,
              lnw_ref, lnb_ref, y_ref):
    e = gid_ref[pl.program_id(0)]
    d_f = d_ref[e].astype(jnp.float32)
    a1 = a1_ref[e]
    a2 = a2_ref[e]

    x = x_ref[...].astype(jnp.bfloat16)
    h = lax.dot_general(x, w1_ref[0], (((1,), (1,)), ((), ())),
                        preferred_element_type=jnp.float32) + b1_ref[0]
    h = jnp.where(h >= 0, h, a1 * h).astype(jnp.bfloat16)
    h = lax.dot_general(h, w2_ref[0], (((1,), (1,)), ((), ())),
                        preferred_element_type=jnp.float32) + b2_ref[0]
    h = jnp.where(h >= 0, h, a2 * h).astype(jnp.bfloat16)
    h = lax.dot_general(h, w3_ref[0], (((1,), (1,)), ((), ())),
                        preferred_element_type=jnp.float32) + b3_ref[0]

    colf = (lax.broadcasted_iota(jnp.int32, (_BB, _DMAX), 1)
            < d_ref[e]).astype(jnp.float32)
    mu = jnp.sum(h * colf, axis=1, keepdims=True) / d_f
    ctr = (h - mu) * colf
    var = jnp.sum(ctr * ctr, axis=1, keepdims=True) / d_f
    y = ctr * lax.rsqrt(var + 1e-5) * lnw_ref[0] + lnb_ref[0]
    y_ref[...] = y * colf


def _grouped_mlp(gid, dvec, a1v, a2v, x_pad, w1, b1, w2, b2, w3, b3, lnw, lnb):
    def wmap(b, gid_ref, *_):
        return (gid_ref[b], 0, 0)

    grid_spec = pltpu.PrefetchScalarGridSpec(
        num_scalar_prefetch=4,
        grid=(_NBLK,),
        in_specs=[
            pl.BlockSpec((_BB, _IN), lambda b, *_: (b, 0)),
            pl.BlockSpec((1, 128, _IN), wmap),
            pl.BlockSpec((1, 1, 128), wmap),
            pl.BlockSpec((1, 64, 128), wmap),
            pl.BlockSpec((1, 1, 64), wmap),
            pl.BlockSpec((1, _DMAX, 64), wmap),
            pl.BlockSpec((1, 1, _DMAX), wmap),
            pl.BlockSpec((1, 1, _DMAX), wmap),
            pl.BlockSpec((1, 1, _DMAX), wmap),
        ],
        out_specs=pl.BlockSpec((_BB, _DMAX), lambda b, *_: (b, 0)),
    )
    return pl.pallas_call(
        _mlp_body,
        grid_spec=grid_spec,
        out_shape=jax.ShapeDtypeStruct((_P, _DMAX), jnp.float32),
    )(gid, dvec, a1v, a2v, x_pad, w1, b1, w2, b2, w3, b3, lnw, lnb)


def _dispatch_body(x_hbm, pos_hbm, xpad_hbm, idx_v, rows_v, sem):
    wid = lax.axis_index("s") * _NC + lax.axis_index("c")
    pltpu.sync_copy(pos_hbm.at[pl.ds(wid * _NCHUNK, _NCHUNK)], idx_v)
    for j in range(_NCHUNK):
        base = wid * _RPW + j * _RCHUNK
        pltpu.sync_copy(x_hbm.at[pl.ds(base, _RCHUNK)], rows_v)
        pltpu.async_copy(rows_v, xpad_hbm.at[idx_v.at[j]], sem).wait()


def _combine_body(ypad_hbm, pos_hbm, out_hbm, idx_v, buf_v, sem):
    wid = lax.axis_index("s") * _NC + lax.axis_index("c")
    pltpu.sync_copy(pos_hbm.at[pl.ds(wid * _NCHUNK, _NCHUNK)], idx_v)
    for j in range(_NCHUNK):
        pltpu.async_copy(ypad_hbm.at[idx_v.at[j]], buf_v, sem).wait()
        base = wid * _RPW + j * _RCHUNK
        pltpu.sync_copy(buf_v, out_hbm.at[pl.ds(base, _RCHUNK)])


@functools.lru_cache(maxsize=1)
def _sc_kernels():
    mesh = plsc.VectorSubcoreMesh(core_axis_name="c", subcore_axis_name="s",
                                  num_cores=_NC, num_subcores=_NS)
    dispatch = pl.kernel(
        _dispatch_body,
        out_type=jax.ShapeDtypeStruct((_P, _IN), jnp.float32),
        mesh=mesh,
        scratch_types=[
            pltpu.VMEM((_NCHUNK, _RCHUNK), jnp.int32),
            pltpu.VMEM((_RCHUNK, _IN), jnp.float32),
            pltpu.SemaphoreType.DMA,
        ],
    )
    combine = pl.kernel(
        _combine_body,
        out_type=jax.ShapeDtypeStruct((_BATCH, _DMAX), jnp.float32),
        mesh=mesh,
        scratch_types=[
            pltpu.VMEM((_NCHUNK, _RCHUNK), jnp.int32),
            pltpu.VMEM((_RCHUNK, _DMAX), jnp.float32),
            pltpu.SemaphoreType.DMA,
        ],
    )
    return dispatch, combine


def kernel(equalized_symbol, csi_context, noise_power, rate_one_hot, params):
    x = jnp.concatenate(
        [equalized_symbol, csi_context, noise_power[:, None],
         jnp.zeros((_BATCH, _IN - _IN_RAW), jnp.float32)], axis=1)
    roh3 = jnp.transpose(rate_one_hot.reshape(128, 128, _NE), (0, 2, 1))

    pos2d, be8 = _route(roh3)
    gid = be8[0, :_NBLK]

    dvec = jnp.array(_LATENTS, jnp.int32)
    a1v = jnp.concatenate([p['a1'] for p in params])
    a2v = jnp.concatenate([p['a2'] for p in params])
    w1 = jnp.stack([jnp.pad(p['W1'], ((0, 0), (0, _IN - _IN_RAW)))
                    for p in params]).astype(jnp.bfloat16)
    b1 = jnp.stack([p['b1'][None] for p in params])
    w2 = jnp.stack([p['W2'] for p in params]).astype(jnp.bfloat16)
    b2 = jnp.stack([p['b2'][None] for p in params])
    w3 = jnp.stack([jnp.pad(p['W3'], ((0, _DMAX - p['W3'].shape[0]), (0, 0)))
                    for p in params]).astype(jnp.bfloat16)
    b3 = jnp.stack([jnp.pad(p['b3'], (0, _DMAX - p['b3'].shape[0]))[None]
                    for p in params])
    lnw = jnp.stack([jnp.pad(p['ln_w'], (0, _DMAX - p['ln_w'].shape[0]))[None]
                     for p in params])
    lnb = jnp.stack([jnp.pad(p['ln_b'], (0, _DMAX - p['ln_b'].shape[0]))[None]
                     for p in params])

    dispatch, combine = _sc_kernels()
    x_pad = dispatch(x, pos2d)
    y_pad = _grouped_mlp(gid, dvec, a1v, a2v, x_pad,
                         w1, b1, w2, b2, w3, b3, lnw, lnb)
    return combine(y_pad, pos2d)


# speedup vs baseline: 1.1046x; 1.1046x over previous
import functools

import jax
import jax.numpy as jnp
from jax import lax
from jax.experimental import pallas as pl
from jax.experimental.pallas import tpu as pltpu
from jax.experimental.pallas import tpu_sc as plsc

_LATENTS = (32, 64, 96, 128, 192, 256)
_NE = 6
_BATCH = 16384
_IN_RAW = 73
_IN = 128
_DMAX = 256
_BB = 2048
_NBLK = _BATCH // _BB + (_NE - 1)
_P = _NBLK * _BB

_NC, _NS = 2, 16
_NW = _NC * _NS
_RPW = _BATCH // _NW
_RCHUNK = 128
_NCHUNK = _RPW // _RCHUNK

_TSTRIP = 1024
_NSTRIP = _BATCH // _TSTRIP


def _routing_body(roh_ref, xt_ref, pos_ref, be_ref, x_ref):
    for m in range(_TSTRIP // 128):
        x_ref[pl.ds(128 * m, 128), :] = xt_ref[:, pl.ds(128 * m, 128)].T

    @pl.when(pl.program_id(0) == 0)
    def _():
        r = [roh_ref[:, e, :] for e in range(_NE)]
        m = r[0]
        for e in range(1, _NE):
            m = jnp.maximum(m, r[e])
        rate = jnp.full((128, 128), _NE, jnp.int32)
        for e in range(_NE - 1, -1, -1):
            rate = jnp.where(r[e] == m, e, rate)

        row_i = lax.broadcasted_iota(jnp.int32, (128, 128), 0)
        col_i = lax.broadcasted_iota(jnp.int32, (128, 128), 1)
        tri_incl = (row_i <= col_i).astype(jnp.float32)
        tri_strict = (col_i < row_i).astype(jnp.float32)

        ohs, ranks, cnts = [], [], []
        for e in range(_NE):
            oh = (rate == e).astype(jnp.float32)
            incl = jnp.dot(oh, tri_incl, preferred_element_type=jnp.float32)
            rowtot = jnp.sum(oh, axis=1, keepdims=True)
            prev = jnp.dot(tri_strict, rowtot, preferred_element_type=jnp.float32)
            ranks.append(incl - oh + prev)
            ohs.append(oh)
            cnts.append(jnp.sum(oh))

        starts = []
        sb = 0
        for e in range(_NE):
            starts.append(sb)
            nb = (cnts[e].astype(jnp.int32) + _BB - 1) // _BB
            sb = sb + nb

        pos = jnp.zeros((128, 128), jnp.float32)
        for e in range(_NE):
            off = (starts[e] * _BB).astype(jnp.float32) if e else jnp.float32(0)
            pos = pos + ohs[e] * (ranks[e] + off)
        pos_ref[...] = pos.astype(jnp.int32)

        bi = lax.broadcasted_iota(jnp.int32, (8, 128), 1)
        be = jnp.full((8, 128), -1, jnp.int32)
        for e in range(_NE):
            se = starts[e] if e else jnp.int32(0)
            be = be + jnp.where(bi >= se, 1, 0)
        be_ref[...] = be


def _route(roh3, xt):
    return pl.pallas_call(
        _routing_body,
        grid=(_NSTRIP,),
        in_specs=[
            pl.BlockSpec((128, _NE, 128), lambda i: (0, 0, 0)),
            pl.BlockSpec((_IN, _TSTRIP), lambda i: (0, i)),
        ],
        out_specs=(
            pl.BlockSpec((128, 128), lambda i: (0, 0)),
            pl.BlockSpec((8, 128), lambda i: (0, 0)),
            pl.BlockSpec((_TSTRIP, _IN), lambda i: (i, 0)),
        ),
        out_shape=(
            jax.ShapeDtypeStruct((128, 128), jnp.int32),
            jax.ShapeDtypeStruct((8, 128), jnp.int32),
            jax.ShapeDtypeStruct((_BATCH, _IN), jnp.float32),
        ),
    )(roh3, xt)


def _mlp_body(gid_ref, d_ref, a1_ref, a2_ref, x_ref,
              w1_ref, b1_ref, w2_ref, b2_ref, w3_ref, b3_ref,
              lnw_ref, lnb_ref, y_ref):
    e = gid_ref[pl.program_id(0)]
    d_f = d_ref[e].astype(jnp.float32)
    a1 = a1_ref[e]
    a2 = a2_ref[e]

    x = x_ref[...].astype(jnp.bfloat16)
    h = lax.dot_general(x, w1_ref[0], (((1,), (1,)), ((), ())),
                        preferred_element_type=jnp.float32) + b1_ref[0]
    h = jnp.where(h >= 0, h, a1 * h).astype(jnp.bfloat16)
    h = lax.dot_general(h, w2_ref[0], (((1,), (1,)), ((), ())),
                        preferred_element_type=jnp.float32) + b2_ref[0]
    h = jnp.where(h >= 0, h, a2 * h).astype(jnp.bfloat16)
    h = lax.dot_general(h, w3_ref[0], (((1,), (1,)), ((), ())),
                        preferred_element_type=jnp.float32) + b3_ref[0]

    colf = (lax.broadcasted_iota(jnp.int32, (_BB, _DMAX), 1)
            < d_ref[e]).astype(jnp.float32)
    mu = jnp.sum(h * colf, axis=1, keepdims=True) / d_f
    ctr = (h - mu) * colf
    var = jnp.sum(ctr * ctr, axis=1, keepdims=True) / d_f
    y_ref[...] = ctr * (lax.rsqrt(var + 1e-5) * lnw_ref[0]) + lnb_ref[0]


def _grouped_mlp(gid, dvec, a1v, a2v, x_pad, w1, b1, w2, b2, w3, b3, lnw, lnb):
    def wmap(b, gid_ref, *_):
        return (gid_ref[b], 0, 0)

    grid_spec = pltpu.PrefetchScalarGridSpec(
        num_scalar_prefetch=4,
        grid=(_NBLK,),
        in_specs=[
            pl.BlockSpec((_BB, _IN), lambda b, *_: (b, 0)),
            pl.BlockSpec((1, 128, _IN), wmap),
            pl.BlockSpec((1, 1, 128), wmap),
            pl.BlockSpec((1, 64, 128), wmap),
            pl.BlockSpec((1, 1, 64), wmap),
            pl.BlockSpec((1, _DMAX, 64), wmap),
            pl.BlockSpec((1, 1, _DMAX), wmap),
            pl.BlockSpec((1, 1, _DMAX), wmap),
            pl.BlockSpec((1, 1, _DMAX), wmap),
        ],
        out_specs=pl.BlockSpec((_BB, _DMAX), lambda b, *_: (b, 0)),
    )
    return pl.pallas_call(
        _mlp_body,
        grid_spec=grid_spec,
        out_shape=jax.ShapeDtypeStruct((_P, _DMAX), jnp.float32),
    )(gid, dvec, a1v, a2v, x_pad, w1, b1, w2, b2, w3, b3, lnw, lnb)


def _dispatch_body(x_hbm, pos_hbm, xpad_hbm, idx_v, rows_v, sem):
    wid = lax.axis_index("s") * _NC + lax.axis_index("c")
    pltpu.sync_copy(pos_hbm.at[pl.ds(wid * _NCHUNK, _NCHUNK)], idx_v)
    for j in range(_NCHUNK):
        base = wid * _RPW + j * _RCHUNK
        pltpu.sync_copy(x_hbm.at[pl.ds(base, _RCHUNK)], rows_v)
        pltpu.async_copy(rows_v, xpad_hbm.at[idx_v.at[j]], sem).wait()


def _combine_body(ypad_hbm, pos_hbm, out_hbm, idx_v, buf_v, sem):
    wid = lax.axis_index("s") * _NC + lax.axis_index("c")
    pltpu.sync_copy(pos_hbm.at[pl.ds(wid * _NCHUNK, _NCHUNK)], idx_v)
    for j in range(_NCHUNK):
        pltpu.async_copy(ypad_hbm.at[idx_v.at[j]], buf_v, sem).wait()
        base = wid * _RPW + j * _RCHUNK
        pltpu.sync_copy(buf_v, out_hbm.at[pl.ds(base, _RCHUNK)])


@functools.lru_cache(maxsize=1)
def _sc_kernels():
    mesh = plsc.VectorSubcoreMesh(core_axis_name="c", subcore_axis_name="s",
                                  num_cores=_NC, num_subcores=_NS)
    dispatch = pl.kernel(
        _dispatch_body,
        out_type=jax.ShapeDtypeStruct((_P, _IN), jnp.float32),
        mesh=mesh,
        scratch_types=[
            pltpu.VMEM((_NCHUNK, _RCHUNK), jnp.int32),
            pltpu.VMEM((_RCHUNK, _IN), jnp.float32),
            pltpu.SemaphoreType.DMA,
        ],
    )
    combine = pl.kernel(
        _combine_body,
        out_type=jax.ShapeDtypeStruct((_BATCH, _DMAX), jnp.float32),
        mesh=mesh,
        scratch_types=[
            pltpu.VMEM((_NCHUNK, _RCHUNK), jnp.int32),
            pltpu.VMEM((_RCHUNK, _DMAX), jnp.float32),
            pltpu.SemaphoreType.DMA,
        ],
    )
    return dispatch, combine


def kernel(equalized_symbol, csi_context, noise_power, rate_one_hot, params):
    xt = jnp.concatenate(
        [equalized_symbol.T, csi_context.T, noise_power[None, :],
         jnp.zeros((_IN - _IN_RAW, _BATCH), jnp.float32)], axis=0)
    roh3 = jnp.transpose(rate_one_hot.reshape(128, 128, _NE), (0, 2, 1))

    pos2d, be8, x = _route(roh3, xt)
    gid = be8[0, :_NBLK]

    dvec = jnp.array(_LATENTS, jnp.int32)
    a1v = jnp.concatenate([p['a1'] for p in params])
    a2v = jnp.concatenate([p['a2'] for p in params])
    w1 = jnp.stack([jnp.pad(p['W1'], ((0, 0), (0, _IN - _IN_RAW)))
                    for p in params]).astype(jnp.bfloat16)
    b1 = jnp.stack([p['b1'][None] for p in params])
    w2 = jnp.stack([p['W2'] for p in params]).astype(jnp.bfloat16)
    b2 = jnp.stack([p['b2'][None] for p in params])
    w3 = jnp.stack([jnp.pad(p['W3'], ((0, _DMAX - p['W3'].shape[0]), (0, 0)))
                    for p in params]).astype(jnp.bfloat16)
    b3 = jnp.stack([jnp.pad(p['b3'], (0, _DMAX - p['b3'].shape[0]))[None]
                    for p in params])
    lnw = jnp.stack([jnp.pad(p['ln_w'], (0, _DMAX - p['ln_w'].shape[0]))[None]
                     for p in params])
    lnb = jnp.stack([jnp.pad(p['ln_b'], (0, _DMAX - p['ln_b'].shape[0]))[None]
                     for p in params])

    dispatch, combine = _sc_kernels()
    x_pad = dispatch(x, pos2d)
    y_pad = _grouped_mlp(gid, dvec, a1v, a2v, x_pad,
                         w1, b1, w2, b2, w3, b3, lnw, lnb)
    return combine(y_pad, pos2d)


# speedup vs baseline: 1.1110x; 1.0058x over previous
import functools

import jax
import jax.numpy as jnp
from jax import lax
from jax.experimental import pallas as pl
from jax.experimental.pallas import tpu as pltpu
from jax.experimental.pallas import tpu_sc as plsc

_LATENTS = (32, 64, 96, 128, 192, 256)
_NE = 6
_BATCH = 16384
_IN_RAW = 73
_IN = 128
_DMAX = 256
_BB = 2048
_NBLK = _BATCH // _BB + (_NE - 1)
_P = _NBLK * _BB

_NC, _NS = 2, 16
_NW = _NC * _NS
_RPW = _BATCH // _NW
_RCHUNK = 128
_NCHUNK = _RPW // _RCHUNK

_TSTRIP = 1024
_NSTRIP = _BATCH // _TSTRIP


def _routing_body(roh_ref, xt_ref, pos_ref, be_ref, x_ref):
    for m in range(_TSTRIP // 128):
        x_ref[pl.ds(128 * m, 128), :] = (
            xt_ref[:, pl.ds(128 * m, 128)].T.astype(jnp.float32))

    @pl.when(pl.program_id(0) == 0)
    def _():
        r = [roh_ref[:, e, :] for e in range(_NE)]
        m = r[0]
        for e in range(1, _NE):
            m = jnp.maximum(m, r[e])
        rate = jnp.full((128, 128), _NE, jnp.int32)
        for e in range(_NE - 1, -1, -1):
            rate = jnp.where(r[e] == m, e, rate)

        row_i = lax.broadcasted_iota(jnp.int32, (128, 128), 0)
        col_i = lax.broadcasted_iota(jnp.int32, (128, 128), 1)
        tri_incl = (row_i <= col_i).astype(jnp.float32)
        tri_strict = (col_i < row_i).astype(jnp.float32)

        ohs, ranks, cnts = [], [], []
        for e in range(_NE):
            oh = (rate == e).astype(jnp.float32)
            incl = jnp.dot(oh, tri_incl, preferred_element_type=jnp.float32)
            rowtot = jnp.sum(oh, axis=1, keepdims=True)
            prev = jnp.dot(tri_strict, rowtot, preferred_element_type=jnp.float32)
            ranks.append(incl - oh + prev)
            ohs.append(oh)
            cnts.append(jnp.sum(oh))

        starts = []
        sb = 0
        for e in range(_NE):
            starts.append(sb)
            nb = (cnts[e].astype(jnp.int32) + _BB - 1) // _BB
            sb = sb + nb

        pos = jnp.zeros((128, 128), jnp.float32)
        for e in range(_NE):
            off = (starts[e] * _BB).astype(jnp.float32) if e else jnp.float32(0)
            pos = pos + ohs[e] * (ranks[e] + off)
        pos_ref[...] = pos.astype(jnp.int32)

        bi = lax.broadcasted_iota(jnp.int32, (8, 128), 1)
        be = jnp.full((8, 128), -1, jnp.int32)
        for e in range(_NE):
            se = starts[e] if e else jnp.int32(0)
            be = be + jnp.where(bi >= se, 1, 0)
        be_ref[...] = be


def _route(roh3, xt):
    return pl.pallas_call(
        _routing_body,
        grid=(_NSTRIP,),
        in_specs=[
            pl.BlockSpec((128, _NE, 128), lambda i: (0, 0, 0)),
            pl.BlockSpec((_IN, _TSTRIP), lambda i: (0, i)),
        ],
        out_specs=(
            pl.BlockSpec((128, 128), lambda i: (0, 0)),
            pl.BlockSpec((8, 128), lambda i: (0, 0)),
            pl.BlockSpec((_TSTRIP, _IN), lambda i: (i, 0)),
        ),
        out_shape=(
            jax.ShapeDtypeStruct((128, 128), jnp.int32),
            jax.ShapeDtypeStruct((8, 128), jnp.int32),
            jax.ShapeDtypeStruct((_BATCH, _IN), jnp.float32),
        ),
    )(roh3, xt)


def _mlp_body(gid_ref, d_ref, a1_ref, a2_ref, x_ref,
              w1_ref, b1_ref, w2_ref, b2_ref, w3_ref, b3_ref,
              lnw_ref, lnb_ref, y_ref):
    e = gid_ref[pl.program_id(0)]
    d_f = d_ref[e].astype(jnp.float32)
    a1 = a1_ref[e]
    a2 = a2_ref[e]

    x = x_ref[...].astype(jnp.bfloat16)
    h = lax.dot_general(x, w1_ref[0], (((1,), (1,)), ((), ())),
                        preferred_element_type=jnp.float32) + b1_ref[0]
    h = jnp.where(h >= 0, h, a1 * h).astype(jnp.bfloat16)
    h = lax.dot_general(h, w2_ref[0], (((1,), (1,)), ((), ())),
                        preferred_element_type=jnp.float32) + b2_ref[0]
    h = jnp.where(h >= 0, h, a2 * h).astype(jnp.bfloat16)
    h = lax.dot_general(h, w3_ref[0], (((1,), (1,)), ((), ())),
                        preferred_element_type=jnp.float32) + b3_ref[0]

    colf = (lax.broadcasted_iota(jnp.int32, (_BB, _DMAX), 1)
            < d_ref[e]).astype(jnp.float32)
    mu = jnp.sum(h * colf, axis=1, keepdims=True) / d_f
    ctr = (h - mu) * colf
    var = jnp.sum(ctr * ctr, axis=1, keepdims=True) / d_f
    y_ref[...] = ctr * (lax.rsqrt(var + 1e-5) * lnw_ref[0]) + lnb_ref[0]


def _grouped_mlp(gid, dvec, a1v, a2v, x_pad, w1, b1, w2, b2, w3, b3, lnw, lnb):
    def wmap(b, gid_ref, *_):
        return (gid_ref[b], 0, 0)

    grid_spec = pltpu.PrefetchScalarGridSpec(
        num_scalar_prefetch=4,
        grid=(_NBLK,),
        in_specs=[
            pl.BlockSpec((_BB, _IN), lambda b, *_: (b, 0)),
            pl.BlockSpec((1, 128, _IN), wmap),
            pl.BlockSpec((1, 1, 128), wmap),
            pl.BlockSpec((1, 64, 128), wmap),
            pl.BlockSpec((1, 1, 64), wmap),
            pl.BlockSpec((1, _DMAX, 64), wmap),
            pl.BlockSpec((1, 1, _DMAX), wmap),
            pl.BlockSpec((1, 1, _DMAX), wmap),
            pl.BlockSpec((1, 1, _DMAX), wmap),
        ],
        out_specs=pl.BlockSpec((_BB, _DMAX), lambda b, *_: (b, 0)),
    )
    return pl.pallas_call(
        _mlp_body,
        grid_spec=grid_spec,
        out_shape=jax.ShapeDtypeStruct((_P, _DMAX), jnp.float32),
    )(gid, dvec, a1v, a2v, x_pad, w1, b1, w2, b2, w3, b3, lnw, lnb)


def _dispatch_body(x_hbm, pos_hbm, xpad_hbm, idx_v, rows_v, sem):
    wid = lax.axis_index("s") * _NC + lax.axis_index("c")
    pltpu.sync_copy(pos_hbm.at[pl.ds(wid * _NCHUNK, _NCHUNK)], idx_v)
    for j in range(_NCHUNK):
        base = wid * _RPW + j * _RCHUNK
        pltpu.sync_copy(x_hbm.at[pl.ds(base, _RCHUNK)], rows_v)
        pltpu.async_copy(rows_v, xpad_hbm.at[idx_v.at[j]], sem).wait()


def _combine_body(ypad_hbm, pos_hbm, out_hbm, idx_v, buf_v, sem):
    wid = lax.axis_index("s") * _NC + lax.axis_index("c")
    pltpu.sync_copy(pos_hbm.at[pl.ds(wid * _NCHUNK, _NCHUNK)], idx_v)
    for j in range(_NCHUNK):
        pltpu.async_copy(ypad_hbm.at[idx_v.at[j]], buf_v, sem).wait()
        base = wid * _RPW + j * _RCHUNK
        pltpu.sync_copy(buf_v, out_hbm.at[pl.ds(base, _RCHUNK)])


@functools.lru_cache(maxsize=1)
def _sc_kernels():
    mesh = plsc.VectorSubcoreMesh(core_axis_name="c", subcore_axis_name="s",
                                  num_cores=_NC, num_subcores=_NS)
    dispatch = pl.kernel(
        _dispatch_body,
        out_type=jax.ShapeDtypeStruct((_P, _IN), jnp.float32),
        mesh=mesh,
        scratch_types=[
            pltpu.VMEM((_NCHUNK, _RCHUNK), jnp.int32),
            pltpu.VMEM((_RCHUNK, _IN), jnp.float32),
            pltpu.SemaphoreType.DMA,
        ],
    )
    combine = pl.kernel(
        _combine_body,
        out_type=jax.ShapeDtypeStruct((_BATCH, _DMAX), jnp.float32),
        mesh=mesh,
        scratch_types=[
            pltpu.VMEM((_NCHUNK, _RCHUNK), jnp.int32),
            pltpu.VMEM((_RCHUNK, _DMAX), jnp.float32),
            pltpu.SemaphoreType.DMA,
        ],
    )
    return dispatch, combine


def kernel(equalized_symbol, csi_context, noise_power, rate_one_hot, params):
    xt = jnp.concatenate(
        [equalized_symbol.T, csi_context.T, noise_power[None, :],
         jnp.zeros((_IN - _IN_RAW, _BATCH), jnp.float32)],
        axis=0).astype(jnp.bfloat16)
    roh3 = jnp.transpose(rate_one_hot.reshape(128, 128, _NE), (0, 2, 1))

    pos2d, be8, x = _route(roh3, xt)
    gid = be8[0, :_NBLK]

    dvec = jnp.array(_LATENTS, jnp.int32)
    a1v = jnp.concatenate([p['a1'] for p in params])
    a2v = jnp.concatenate([p['a2'] for p in params])
    w1 = jnp.stack([jnp.pad(p['W1'], ((0, 0), (0, _IN - _IN_RAW)))
                    for p in params]).astype(jnp.bfloat16)
    b1 = jnp.stack([p['b1'][None] for p in params])
    w2 = jnp.stack([p['W2'] for p in params]).astype(jnp.bfloat16)
    b2 = jnp.stack([p['b2'][None] for p in params])
    w3 = jnp.stack([jnp.pad(p['W3'], ((0, _DMAX - p['W3'].shape[0]), (0, 0)))
                    for p in params]).astype(jnp.bfloat16)
    b3 = jnp.stack([jnp.pad(p['b3'], (0, _DMAX - p['b3'].shape[0]))[None]
                    for p in params])
    lnw = jnp.stack([jnp.pad(p['ln_w'], (0, _DMAX - p['ln_w'].shape[0]))[None]
                     for p in params])
    lnb = jnp.stack([jnp.pad(p['ln_b'], (0, _DMAX - p['ln_b'].shape[0]))[None]
                     for p in params])

    dispatch, combine = _sc_kernels()
    x_pad = dispatch(x, pos2d)
    y_pad = _grouped_mlp(gid, dvec, a1v, a2v, x_pad,
                         w1, b1, w2, b2, w3, b3, lnw, lnb)
    return combine(y_pad, pos2d)
